# baseline (speedup 1.0000x reference)
import jax
import jax.numpy as jnp
from jax.experimental import pallas as pl

_INPUT_SIZE = 768
_LATENT = 64
_LEVELS = 4
_K = 1024
_B = 16384
_BT = 512


def _fused_body(x_ref, we0, be0, we1, be1, we2, be2, cb_ref, cn_ref,
                wd0, bd0, wd1, bd1, wd2, bd2,
                dec_ref, r_ref, e_ref, idx_ref, cnt_ref):
    f32 = jnp.float32
    h = jnp.dot(x_ref[...], we0[...], preferred_element_type=f32, precision=jax.lax.Precision.DEFAULT) + be0[...]
    h = jnp.maximum(h, 0.0)
    h = jnp.dot(h, we1[...], preferred_element_type=f32, precision=jax.lax.Precision.DEFAULT) + be1[...]
    h = jnp.maximum(h, 0.0)
    z = jnp.dot(h, we2[...], preferred_element_type=f32, precision=jax.lax.Precision.DEFAULT) + be2[...]

    residual = z
    z_hat = jnp.zeros_like(z)
    idx_cols, cnt_rows = [], []
    iota_k = jax.lax.broadcasted_iota(jnp.int32, (_BT, _K), 1)
    ones_row = jnp.ones((1, _LATENT), dtype=f32)
    for l in range(_LEVELS):
        cb = cb_ref[l]
        cnorm_row = cn_ref[l]
        scores = jax.lax.dot_general(
            residual, cb, (((1,), (1,)), ((), ())),
            precision=jax.lax.Precision.DEFAULT)
        rnorm = jnp.sum(residual * residual, axis=1, keepdims=True)
        d = (rnorm - 2.0 * scores) + cnorm_row
        dmin = jnp.min(d, axis=1, keepdims=True)
        idx2d = jnp.min(jnp.where(d <= dmin, iota_k, _K), axis=1,
                        keepdims=True)
        onehot = (iota_k == idx2d).astype(f32)
        e_l = jnp.dot(onehot, cb, preferred_element_type=f32, precision=jax.lax.Precision.HIGHEST)
        r_ref[l] = residual
        e_ref[l] = e_l
        idx_cols.append(idx2d)
        cnt_rows.append(jnp.sum(onehot, axis=0, keepdims=True))
        z_hat = z_hat + e_l
        residual = residual - e_l

    idx_ref[...] = jnp.concatenate(idx_cols, axis=1)

    @pl.when(pl.program_id(0) == 0)
    def _init():
        cnt_ref[...] = jnp.zeros_like(cnt_ref)

    cnt_ref[...] += jnp.concatenate(cnt_rows, axis=0)

    h = jnp.dot(z_hat, wd0[...], preferred_element_type=f32, precision=jax.lax.Precision.DEFAULT) + bd0[...]
    h = jnp.maximum(h, 0.0)
    h = jnp.dot(h, wd1[...], preferred_element_type=f32, precision=jax.lax.Precision.DEFAULT) + bd1[...]
    h = jnp.maximum(h, 0.0)
    dec_ref[...] = jnp.dot(h, wd2[...], preferred_element_type=f32, precision=jax.lax.Precision.DEFAULT) + bd2[...]


@jax.jit
def kernel(x, We0, be0, We1, be1, We2, be2, codebooks,
           Wd0, bd0, Wd1, bd1, Wd2, bd2):
    nb = _B // _BT
    cb_norms = jnp.sum(codebooks * codebooks, axis=2)[:, None, :]
    full = lambda shape: pl.BlockSpec(shape, lambda i: (0,) * len(shape))
    out = pl.pallas_call(
        _fused_body,
        grid=(nb,),
        in_specs=[
            pl.BlockSpec((_BT, _INPUT_SIZE), lambda i: (i, 0)),
            full(We0.shape), full((1, 512)),
            full(We1.shape), full((1, 256)),
            full(We2.shape), full((1, _LATENT)),
            full(codebooks.shape),
            full((_LEVELS, 1, _K)),
            full(Wd0.shape), full((1, 256)),
            full(Wd1.shape), full((1, 512)),
            full(Wd2.shape), full((1, _INPUT_SIZE)),
        ],
        out_specs=[
            pl.BlockSpec((_BT, _INPUT_SIZE), lambda i: (i, 0)),
            pl.BlockSpec((_LEVELS, _BT, _LATENT), lambda i: (0, i, 0)),
            pl.BlockSpec((_LEVELS, _BT, _LATENT), lambda i: (0, i, 0)),
            pl.BlockSpec((_BT, _LEVELS), lambda i: (i, 0)),
            pl.BlockSpec((_LEVELS, _K), lambda i: (0, 0)),
        ],
        out_shape=[
            jax.ShapeDtypeStruct((_B, _INPUT_SIZE), jnp.float32),
            jax.ShapeDtypeStruct((_LEVELS, _B, _LATENT), jnp.float32),
            jax.ShapeDtypeStruct((_LEVELS, _B, _LATENT), jnp.float32),
            jax.ShapeDtypeStruct((_B, _LEVELS), jnp.int32),
            jax.ShapeDtypeStruct((_LEVELS, _K), jnp.float32),
        ],
    )(x, We0, be0.reshape(1, -1), We1, be1.reshape(1, -1),
      We2, be2.reshape(1, -1), codebooks, cb_norms,
      Wd0, bd0.reshape(1, -1), Wd1, bd1.reshape(1, -1), Wd2, bd2.reshape(1, -1))
    decoded, r, e, quantized, counts_f = out
    return (decoded, r, e, counts_f.astype(jnp.int32), quantized)
